# baseline (speedup 1.0000x reference)
import functools

import jax
import jax.numpy as jnp
from jax import lax
from jax.experimental import pallas as pl
from jax.experimental.pallas import tpu as pltpu
from jax.experimental.pallas import tpu_sc as plsc

_NC = 2
_NS = 16
_NW = _NC * _NS
_L = 16


def _make_sc_call(M, B, C):
    E = M // _NW
    n_chunks = E // C
    n_grp = C // _L

    mesh = plsc.VectorSubcoreMesh(
        core_axis_name="c", subcore_axis_name="s",
        num_cores=_NC, num_subcores=_NS)

    @functools.partial(
        pl.kernel,
        out_type=[jax.ShapeDtypeStruct((M,), jnp.float32)] * 3,
        mesh=mesh,
        compiler_params=pltpu.CompilerParams(
            needs_layout_passes=False, use_tc_tiling_on_sc=False),
        scratch_types=[
            pltpu.VMEM((B * 9,), jnp.float32),
            pltpu.VMEM((C,), jnp.float32),
            pltpu.VMEM((C,), jnp.float32),
            pltpu.VMEM((C,), jnp.float32),
            pltpu.VMEM((C,), jnp.int32),
            pltpu.VMEM((C,), jnp.float32),
            pltpu.VMEM((C,), jnp.float32),
            pltpu.VMEM((C,), jnp.float32),
        ],
    )
    def sc_call(e0_hbm, e1_hbm, e2_hbm, bid_hbm, lat_hbm,
                s0_hbm, s1_hbm, s2_hbm,
                lat_v, w0_v, w1_v, w2_v, bid_v, s0_v, s1_v, s2_v):
        wid = lax.axis_index("s") * _NC + lax.axis_index("c")
        base_e = wid * E
        pltpu.sync_copy(lat_hbm, lat_v)
        ei_hbm = (e0_hbm, e1_hbm, e2_hbm)
        w_v = (w0_v, w1_v, w2_v)
        s_hbm = (s0_hbm, s1_hbm, s2_hbm)
        s_v = (s0_v, s1_v, s2_v)

        def chunk_body(ci, _):
            e0 = base_e + ci * C
            for i in range(3):
                pltpu.sync_copy(ei_hbm[i].at[pl.ds(e0, C)], w_v[i])
            pltpu.sync_copy(bid_hbm.at[pl.ds(e0, C)], bid_v)

            @plsc.parallel_loop(0, C, _L, unroll=8)
            def _grp(gl):
                sl = pl.ds(gl, _L)
                bid16 = bid_v[sl]
                bid16 = jnp.minimum(jnp.maximum(bid16, 0), B - 1)
                lbase = bid16 * 9
                w0 = w0_v[sl]
                w1 = w1_v[sl]
                w2 = w2_v[sl]
                for j in range(3):
                    l0 = plsc.load_gather(lat_v, [lbase + j])
                    l1 = plsc.load_gather(lat_v, [lbase + (3 + j)])
                    l2 = plsc.load_gather(lat_v, [lbase + (6 + j)])
                    s_v[j][sl] = w0 * l0 + w1 * l1 + w2 * l2
            for j in range(3):
                pltpu.sync_copy(s_v[j], s_hbm[j].at[pl.ds(e0, C)])
            return 0

        lax.fori_loop(0, n_chunks, chunk_body, 0)

    return sc_call


def kernel(position, edge_image, lattice, batch_id_edge):
    M = position.shape[0]
    B = lattice.shape[0]
    assert M % _NW == 0
    C = 8000
    assert (M // _NW) % C == 0 and C % _L == 0

    ei0 = edge_image[:, 0].astype(jnp.float32)
    ei1 = edge_image[:, 1].astype(jnp.float32)
    ei2 = edge_image[:, 2].astype(jnp.float32)
    lat_f = lattice.astype(jnp.float32).reshape(B * 9)

    s0, s1, s2 = _make_sc_call(M, B, C)(
        ei0, ei1, ei2, batch_id_edge.astype(jnp.int32), lat_f)
    return position + jnp.stack([s0, s1, s2], axis=-1)


# speedup vs baseline: 1.4825x; 1.4825x over previous
import functools

import jax
import jax.numpy as jnp
from jax import lax
from jax.experimental import pallas as pl
from jax.experimental.pallas import tpu as pltpu
from jax.experimental.pallas import tpu_sc as plsc

_NC = 2
_NS = 16
_NW = _NC * _NS
_L = 16


def _make_sc_call(M, B, C):
    E = M // _NW
    n_chunks = E // C
    assert n_chunks % 2 == 0 and n_chunks >= 4

    mesh = plsc.VectorSubcoreMesh(
        core_axis_name="c", subcore_axis_name="s",
        num_cores=_NC, num_subcores=_NS)

    @functools.partial(
        pl.kernel,
        out_type=[jax.ShapeDtypeStruct((M,), jnp.float32)] * 3,
        mesh=mesh,
        compiler_params=pltpu.CompilerParams(
            needs_layout_passes=False, use_tc_tiling_on_sc=False),
        scratch_types=[
            pltpu.VMEM((B * 9,), jnp.float32),
            [[pltpu.VMEM((C,), jnp.float32)] * 3] * 2,
            [pltpu.VMEM((C,), jnp.int32)] * 2,
            [[pltpu.VMEM((C,), jnp.float32)] * 3] * 2,
            [pltpu.SemaphoreType.DMA] * 2,
            [pltpu.SemaphoreType.DMA] * 2,
        ],
    )
    def sc_call(e0_hbm, e1_hbm, e2_hbm, bid_hbm, lat_hbm,
                s0_hbm, s1_hbm, s2_hbm,
                lat_v, w_v, bid_v, s_v, sem_in, sem_out):
        wid = lax.axis_index("s") * _NC + lax.axis_index("c")
        base_e = wid * E
        pltpu.sync_copy(lat_hbm, lat_v)
        ei_hbm = (e0_hbm, e1_hbm, e2_hbm)
        s_hbm = (s0_hbm, s1_hbm, s2_hbm)

        def in_copies(b, ci):
            e0 = base_e + ci * C
            cps = [pltpu.make_async_copy(
                ei_hbm[i].at[pl.ds(e0, C)], w_v[b][i], sem_in[b])
                for i in range(3)]
            cps.append(pltpu.make_async_copy(
                bid_hbm.at[pl.ds(e0, C)], bid_v[b], sem_in[b]))
            return cps

        def out_copies(b, ci):
            e0 = base_e + ci * C
            return [pltpu.make_async_copy(
                s_v[b][j], s_hbm[j].at[pl.ds(e0, C)], sem_out[b])
                for j in range(3)]

        def compute(b):
            @plsc.parallel_loop(0, C, _L, unroll=4)
            def _grp(gl):
                sl = pl.ds(gl, _L)
                bid16 = bid_v[b][sl]
                bid16 = jnp.minimum(jnp.maximum(bid16, 0), B - 1)
                lbase = bid16 * 9
                w0 = w_v[b][0][sl]
                w1 = w_v[b][1][sl]
                w2 = w_v[b][2][sl]
                for j in range(3):
                    l0 = plsc.load_gather(lat_v, [lbase + j])
                    l1 = plsc.load_gather(lat_v, [lbase + (3 + j)])
                    l2 = plsc.load_gather(lat_v, [lbase + (6 + j)])
                    s_v[b][j][sl] = w0 * l0 + w1 * l1 + w2 * l2

        def do_chunk(b, ci, drain_prev_out):
            for cp in in_copies(b, ci):
                cp.wait()
            @pl.when(ci + 1 < n_chunks)
            def _():
                for cp in in_copies(1 - b, ci + 1):
                    cp.start()
            if drain_prev_out:
                for cp in out_copies(b, ci - 2):
                    cp.wait()
            compute(b)
            for cp in out_copies(b, ci):
                cp.start()

        for cp in in_copies(0, 0):
            cp.start()
        do_chunk(0, 0, False)
        do_chunk(1, 1, False)

        def pair_body(k, _):
            do_chunk(0, 2 * k, True)
            do_chunk(1, 2 * k + 1, True)
            return 0

        lax.fori_loop(1, n_chunks // 2, pair_body, 0)
        for b in (0, 1):
            for cp in out_copies(b, n_chunks - 2 + b):
                cp.wait()

    return sc_call


def kernel(position, edge_image, lattice, batch_id_edge):
    M = position.shape[0]
    B = lattice.shape[0]
    assert M % _NW == 0
    C = 4000
    assert (M // _NW) % C == 0 and C % _L == 0

    ei0 = edge_image[:, 0].astype(jnp.float32)
    ei1 = edge_image[:, 1].astype(jnp.float32)
    ei2 = edge_image[:, 2].astype(jnp.float32)
    lat_f = lattice.astype(jnp.float32).reshape(B * 9)

    s0, s1, s2 = _make_sc_call(M, B, C)(
        ei0, ei1, ei2, batch_id_edge.astype(jnp.int32), lat_f)
    return position + jnp.stack([s0, s1, s2], axis=-1)
